# baseline (speedup 1.0000x reference)
import functools

import jax
import jax.numpy as jnp
from jax import lax
from jax.experimental import pallas as pl
from jax.experimental.pallas import tpu as pltpu
from jax.experimental.pallas import tpu_sc as plsc

_EPS = 1e-12
_L = 16


def _lane_shuffle(x, idx):
    return lax.gather(
        x, idx[:, None],
        dimension_numbers=lax.GatherDimensionNumbers(
            offset_dims=(), collapsed_slice_dims=(0,), start_index_map=(0,)),
        slice_sizes=(1,),
        mode=lax.GatherScatterMode.PROMISE_IN_BOUNDS)


def _make_sc_kernel(n_tasks, d, batch):
    info = plsc.get_sparse_core_info()
    nc, ns = info.num_cores, info.num_subcores
    nw = nc * ns
    assert batch % nw == 0
    b_per_w = batch // nw
    chunk = min(128, b_per_w)
    n_chunks = b_per_w // chunk
    mesh = plsc.VectorSubcoreMesh(core_axis_name="c", subcore_axis_name="s")

    @functools.partial(
        pl.kernel,
        out_type=jax.ShapeDtypeStruct((d, batch), jnp.float32),
        mesh=mesh,
        scratch_types=[
            pltpu.VMEM((n_chunks, chunk), jnp.int32),
            pltpu.VMEM((b_per_w, d), jnp.float32),
            pltpu.SemaphoreType.DMA,
            pltpu.SemaphoreType.DMA,
        ],
    )
    def sc_kernel(table_hbm, tasks_hbm, out_hbm, idx_v, rows_v, gsem, ssem):
        wid = lax.axis_index("s") * nc + lax.axis_index("c")
        base = wid * b_per_w

        for j in range(n_chunks):
            pltpu.sync_copy(tasks_hbm.at[pl.ds(base + j * chunk, chunk)],
                            idx_v.at[j])
        gathers = [
            pltpu.async_copy(table_hbm.at[idx_v.at[j]],
                             rows_v.at[pl.ds(j * chunk, chunk)], gsem)
            for j in range(n_chunks)
        ]

        iota = lax.iota(jnp.int32, _L)
        perms = [iota ^ k for k in (1, 2, 4)]

        def pair_body(rr, carry):
            for p in range(2):
                r = rr * 2 + p
                for j in range(d // _L):
                    x = rows_v[r, pl.ds(j * _L, _L)]
                    sig = 1.0 / (1.0 + jnp.exp(-x))
                    t = sig
                    for pm in perms:
                        t = t + _lane_shuffle(t, pm)
                    rows_v[r, pl.ds(j * _L, _L)] = sig / (t + _EPS)
            return carry

        for j in range(n_chunks):
            gathers[j].wait()
            lax.fori_loop(j * (chunk // 2), (j + 1) * (chunk // 2),
                          pair_body, 0)
        scatters = [
            pltpu.async_copy(rows_v.at[:, c],
                             out_hbm.at[c, pl.ds(base, b_per_w)], ssem)
            for c in range(d)
        ]
        for s in scatters:
            s.wait()

    return sc_kernel


@jax.jit
def kernel(module_logits, tasks):
    n_tasks, d = module_logits.shape
    batch = tasks.shape[0]
    fn = _make_sc_kernel(n_tasks, d, batch)
    out_cb = fn(module_logits, tasks.astype(jnp.int32))
    return out_cb.reshape(d // 8, 8, batch).transpose(2, 0, 1)


# speedup vs baseline: 97.5812x; 97.5812x over previous
import functools

import jax
import jax.numpy as jnp
from jax import lax
from jax.experimental import pallas as pl
from jax.experimental.pallas import tpu as pltpu
from jax.experimental.pallas import tpu_sc as plsc

_EPS = 1e-12
_L = 16


def _lane_shuffle(x, idx):
    return lax.gather(
        x, idx[:, None],
        dimension_numbers=lax.GatherDimensionNumbers(
            offset_dims=(), collapsed_slice_dims=(0,), start_index_map=(0,)),
        slice_sizes=(1,),
        mode=lax.GatherScatterMode.PROMISE_IN_BOUNDS)


def _make_sc_kernel(n_tasks, d, batch):
    info = plsc.get_sparse_core_info()
    nc, ns = info.num_cores, info.num_subcores
    nw = nc * ns
    assert batch % nw == 0
    b_per_w = batch // nw
    chunk = min(128, b_per_w)
    n_chunks = b_per_w // chunk
    mesh = plsc.VectorSubcoreMesh(core_axis_name="c", subcore_axis_name="s")

    @functools.partial(
        pl.kernel,
        out_type=jax.ShapeDtypeStruct((batch, d), jnp.float32),
        mesh=mesh,
        scratch_types=[
            pltpu.VMEM((n_chunks, chunk), jnp.int32),
            pltpu.VMEM((b_per_w, d), jnp.float32),
            pltpu.SemaphoreType.DMA,
            pltpu.SemaphoreType.DMA,
        ],
    )
    def sc_kernel(table_hbm, tasks_hbm, out_hbm, idx_v, rows_v, gsem, ssem):
        wid = lax.axis_index("s") * nc + lax.axis_index("c")
        base = wid * b_per_w

        for j in range(n_chunks):
            pltpu.sync_copy(tasks_hbm.at[pl.ds(base + j * chunk, chunk)],
                            idx_v.at[j])
        gathers = [
            pltpu.async_copy(table_hbm.at[idx_v.at[j]],
                             rows_v.at[pl.ds(j * chunk, chunk)], gsem)
            for j in range(n_chunks)
        ]

        iota = lax.iota(jnp.int32, _L)
        perms = [iota ^ k for k in (1, 2, 4)]

        def compute_chunk(jc):
            @plsc.parallel_loop(jc * chunk, (jc + 1) * chunk, unroll=2)
            def row_body(r):
                for j in range(d // _L):
                    x = rows_v[r, pl.ds(j * _L, _L)]
                    sig = 1.0 / (1.0 + jnp.exp(-x))
                    t = sig
                    for pm in perms:
                        t = t + _lane_shuffle(t, pm)
                    rows_v[r, pl.ds(j * _L, _L)] = sig / (t + _EPS)

        scatters = []
        for j in range(n_chunks):
            gathers[j].wait()
            compute_chunk(j)
            scatters.append(
                pltpu.async_copy(rows_v.at[pl.ds(j * chunk, chunk)],
                                 out_hbm.at[pl.ds(base + j * chunk, chunk)],
                                 ssem))
        for s in scatters:
            s.wait()

    return sc_kernel


@jax.jit
def kernel(module_logits, tasks):
    n_tasks, d = module_logits.shape
    batch = tasks.shape[0]
    fn = _make_sc_kernel(n_tasks, d, batch)
    out = fn(module_logits, tasks.astype(jnp.int32))
    return out.reshape(batch, d // 8, 8)
